# Initial kernel scaffold; baseline (speedup 1.0000x reference)
import jax
import jax.numpy as jnp
from jax.experimental import pallas as pl


def kernel(x, Wg, W1, b1, W2, b2):
    raise NotImplementedError("write your pallas kernel here")


# speedup vs baseline: 1.2603x; 1.2603x over previous
import functools

import jax
import jax.numpy as jnp
from jax import lax
from jax.experimental import pallas as pl
from jax.experimental.pallas import tpu as pltpu
from jax.experimental.pallas import tpu_sc as plsc

N, D, E, K, F = 2048, 1024, 8, 2, 2048
TEMP = 1.0

BT = 256
BLK = 128
G = 40
RPAD = G * BLK


def _router_body(x_ref, wg_ref, probs_ref, topv_ref, topi_ref, wn_ref,
                 ent_ref, mass_ref):
    l = jnp.dot(x_ref[...], wg_ref[...], preferred_element_type=jnp.float32)
    l = l / TEMP
    m = jnp.max(l, axis=1, keepdims=True)
    e = jnp.exp(l - m)
    s = jnp.sum(e, axis=1, keepdims=True)
    p = e / s
    probs_ref[...] = p
    iota = lax.broadcasted_iota(jnp.int32, p.shape, 1)
    v1 = jnp.max(p, axis=1, keepdims=True)
    i1 = jnp.min(jnp.where(p == v1, iota, E), axis=1, keepdims=True)
    p2 = jnp.where(iota == i1, -1.0, p)
    v2 = jnp.max(p2, axis=1, keepdims=True)
    i2 = jnp.min(jnp.where(p2 == v2, iota, E), axis=1, keepdims=True)
    topv_ref[...] = jnp.concatenate([v1, v2], axis=1)
    topi_ref[...] = jnp.concatenate([i1, i2], axis=1)
    ws = v1 + v2
    wn_ref[...] = jnp.concatenate([v1 / ws, v2 / ws], axis=1)
    ent_ref[...] = -jnp.sum(p * jnp.log(p + 1e-9), axis=1, keepdims=True)
    mass_ref[...] = ws


def _router(x, Wg):
    return pl.pallas_call(
        _router_body,
        grid=(N // BT,),
        in_specs=[
            pl.BlockSpec((BT, D), lambda i: (i, 0)),
            pl.BlockSpec((D, E), lambda i: (0, 0)),
        ],
        out_specs=[
            pl.BlockSpec((BT, E), lambda i: (i, 0)),
            pl.BlockSpec((BT, K), lambda i: (i, 0)),
            pl.BlockSpec((BT, K), lambda i: (i, 0)),
            pl.BlockSpec((BT, K), lambda i: (i, 0)),
            pl.BlockSpec((BT, 1), lambda i: (i, 0)),
            pl.BlockSpec((BT, 1), lambda i: (i, 0)),
        ],
        out_shape=[
            jax.ShapeDtypeStruct((N, E), jnp.float32),
            jax.ShapeDtypeStruct((N, K), jnp.float32),
            jax.ShapeDtypeStruct((N, K), jnp.int32),
            jax.ShapeDtypeStruct((N, K), jnp.float32),
            jax.ShapeDtypeStruct((N, 1), jnp.float32),
            jax.ShapeDtypeStruct((N, 1), jnp.float32),
        ],
    )(x, Wg)


def _ffn_body(gid_ref, nblk_ref, xs_ref, w1_ref, b1_ref, w2_ref, b2_ref,
              w_ref, ys_ref):
    g = pl.program_id(0)

    @pl.when(g < nblk_ref[0])
    def _():
        h = jnp.dot(xs_ref[...], w1_ref[0], preferred_element_type=jnp.float32)
        h = jnp.maximum(h + b1_ref[0], 0.0)
        y = jnp.dot(h, w2_ref[0], preferred_element_type=jnp.float32)
        ys_ref[...] = (y + b2_ref[0]) * w_ref[...]


def _ffn(xs, W1, b1, W2, b2, w_col, gid, nblk):
    grid_spec = pltpu.PrefetchScalarGridSpec(
        num_scalar_prefetch=2,
        grid=(G,),
        in_specs=[
            pl.BlockSpec((BLK, D), lambda g, gid, nblk: (g, 0)),
            pl.BlockSpec((1, D, F), lambda g, gid, nblk: (gid[g], 0, 0)),
            pl.BlockSpec((1, 1, F), lambda g, gid, nblk: (gid[g], 0, 0)),
            pl.BlockSpec((1, F, D), lambda g, gid, nblk: (gid[g], 0, 0)),
            pl.BlockSpec((1, 1, D), lambda g, gid, nblk: (gid[g], 0, 0)),
            pl.BlockSpec((BLK, 1), lambda g, gid, nblk: (g, 0)),
        ],
        out_specs=pl.BlockSpec((BLK, D), lambda g, gid, nblk: (g, 0)),
    )
    return pl.pallas_call(
        _ffn_body,
        grid_spec=grid_spec,
        out_shape=jax.ShapeDtypeStruct((RPAD, D), jnp.float32),
    )(gid, nblk, xs, W1, b1.reshape(E, 1, F), W2, b2.reshape(E, 1, D), w_col)


_NC, _NS = 2, 16
_NW = _NC * _NS

_G_RPW = RPAD // _NW
_G_CH = 80

_C_TPW = N // _NW
_C_CH = 32

@functools.cache
def _sc_kernels():
    mesh = plsc.VectorSubcoreMesh(
        core_axis_name="c", subcore_axis_name="s", num_cores=_NC)

    @functools.partial(
        pl.kernel,
        mesh=mesh,
        out_type=jax.ShapeDtypeStruct((RPAD, D), jnp.float32),
        scratch_types=[
            pltpu.VMEM((_G_CH,), jnp.int32),
            pltpu.VMEM((_G_CH, D), jnp.float32),
            pltpu.SemaphoreType.DMA,
        ],
    )
    def sc_gather(tok_hbm, x_hbm, xs_hbm, idx_v, rows_v, sem):
        wid = lax.axis_index("s") * _NC + lax.axis_index("c")
        for t in range(_G_RPW // _G_CH):
            base = wid * _G_RPW + t * _G_CH
            pltpu.sync_copy(tok_hbm.at[pl.ds(base, _G_CH)], idx_v)
            pltpu.async_copy(x_hbm.at[idx_v], rows_v, sem).wait()
            pltpu.sync_copy(rows_v, xs_hbm.at[pl.ds(base, _G_CH)])

    @functools.partial(
        pl.kernel,
        mesh=mesh,
        out_type=jax.ShapeDtypeStruct((N, D), jnp.float32),
        scratch_types=[
            pltpu.VMEM((_C_CH,), jnp.int32),
            pltpu.VMEM((_C_CH,), jnp.int32),
            pltpu.VMEM((_C_CH, D), jnp.float32),
            pltpu.VMEM((_C_CH, D), jnp.float32),
            pltpu.SemaphoreType.DMA,
            pltpu.SemaphoreType.DMA,
        ],
    )
    def sc_combine(p0_hbm, p1_hbm, ys_hbm, out_hbm,
                   i0_v, i1_v, a_v, b_v, s0, s1):
        wid = lax.axis_index("s") * _NC + lax.axis_index("c")
        for t in range(_C_TPW // _C_CH):
            base = wid * _C_TPW + t * _C_CH
            pltpu.sync_copy(p0_hbm.at[pl.ds(base, _C_CH)], i0_v)
            pltpu.sync_copy(p1_hbm.at[pl.ds(base, _C_CH)], i1_v)
            cp0 = pltpu.async_copy(ys_hbm.at[i0_v], a_v, s0)
            cp1 = pltpu.async_copy(ys_hbm.at[i1_v], b_v, s1)
            cp0.wait()
            cp1.wait()

            def _row(r, carry):
                def _col(c, cc):
                    sl = pl.ds(c * 16, 16)
                    a_v[r, sl] = a_v[r, sl] + b_v[r, sl]
                    return cc
                return lax.fori_loop(0, D // 16, _col, carry, unroll=4)

            lax.fori_loop(0, _C_CH, _row, 0)
            pltpu.sync_copy(a_v, out_hbm.at[pl.ds(base, _C_CH)])

    return sc_gather, sc_combine


def _dispatch(topi, wn):
    e_flat = topi.reshape(-1)
    w_flat = wn.reshape(-1)
    onehot = (e_flat[:, None] == jnp.arange(E, dtype=e_flat.dtype))
    oh_i = onehot.astype(jnp.int32)
    counts = jnp.sum(oh_i, axis=0)
    ranks = jnp.sum(jnp.where(onehot, jnp.cumsum(oh_i, axis=0) - 1, 0), axis=1)
    nb = (counts + BLK - 1) // BLK
    cum_nb = jnp.cumsum(nb)
    rowstart = BLK * (cum_nb - nb)
    dst = rowstart[e_flat] + ranks
    tok = jnp.zeros((RPAD,), jnp.int32).at[dst].set(
        jnp.arange(N * K, dtype=jnp.int32) // K)
    w_pad = jnp.zeros((RPAD,), jnp.float32).at[dst].set(w_flat)
    gid = jnp.minimum(
        jnp.searchsorted(cum_nb, jnp.arange(G, dtype=jnp.int32), side="right"),
        E - 1).astype(jnp.int32)
    nblk = cum_nb[E - 1:].astype(jnp.int32)
    p = dst.reshape(N, K)
    return tok, w_pad, gid, nblk, p[:, 0], p[:, 1]


def kernel(x, Wg, W1, b1, W2, b2):
    probs, topv, topi, wn, ent, mass = _router(x, Wg)
    tok, w_pad, gid, nblk, p0, p1 = _dispatch(topi, wn)
    sc_gather, sc_combine = _sc_kernels()
    xs = sc_gather(tok, x)
    ys = _ffn(xs, W1, b1, W2, b2, w_pad.reshape(RPAD, 1), gid, nblk)
    out = sc_combine(p0, p1, ys)
    return (out, probs, topi, topv, ent.reshape(N), mass.reshape(N))


# speedup vs baseline: 1.2625x; 1.0017x over previous
import functools

import jax
import jax.numpy as jnp
from jax import lax
from jax.experimental import pallas as pl
from jax.experimental.pallas import tpu as pltpu
from jax.experimental.pallas import tpu_sc as plsc

N, D, E, K, F = 2048, 1024, 8, 2, 2048
TEMP = 1.0

BT = 256
BLK = 128
G = 40
RPAD = G * BLK


def _router_body(x_ref, wg_ref, probs_ref, topv_ref, topi_ref, wn_ref,
                 ent_ref, mass_ref):
    l = jnp.dot(x_ref[...], wg_ref[...], preferred_element_type=jnp.float32)
    l = l / TEMP
    m = jnp.max(l, axis=1, keepdims=True)
    e = jnp.exp(l - m)
    s = jnp.sum(e, axis=1, keepdims=True)
    p = e / s
    probs_ref[...] = p
    iota = lax.broadcasted_iota(jnp.int32, p.shape, 1)
    v1 = jnp.max(p, axis=1, keepdims=True)
    i1 = jnp.min(jnp.where(p == v1, iota, E), axis=1, keepdims=True)
    p2 = jnp.where(iota == i1, -1.0, p)
    v2 = jnp.max(p2, axis=1, keepdims=True)
    i2 = jnp.min(jnp.where(p2 == v2, iota, E), axis=1, keepdims=True)
    topv_ref[...] = jnp.concatenate([v1, v2], axis=1)
    topi_ref[...] = jnp.concatenate([i1, i2], axis=1)
    ws = v1 + v2
    wn_ref[...] = jnp.concatenate([v1 / ws, v2 / ws], axis=1)
    ent_ref[...] = -jnp.sum(p * jnp.log(p + 1e-9), axis=1, keepdims=True)
    mass_ref[...] = ws


def _router(x, Wg):
    return pl.pallas_call(
        _router_body,
        grid=(N // BT,),
        in_specs=[
            pl.BlockSpec((BT, D), lambda i: (i, 0)),
            pl.BlockSpec((D, E), lambda i: (0, 0)),
        ],
        out_specs=[
            pl.BlockSpec((BT, E), lambda i: (i, 0)),
            pl.BlockSpec((BT, K), lambda i: (i, 0)),
            pl.BlockSpec((BT, K), lambda i: (i, 0)),
            pl.BlockSpec((BT, K), lambda i: (i, 0)),
            pl.BlockSpec((BT, 1), lambda i: (i, 0)),
            pl.BlockSpec((BT, 1), lambda i: (i, 0)),
        ],
        out_shape=[
            jax.ShapeDtypeStruct((N, E), jnp.float32),
            jax.ShapeDtypeStruct((N, K), jnp.float32),
            jax.ShapeDtypeStruct((N, K), jnp.int32),
            jax.ShapeDtypeStruct((N, K), jnp.float32),
            jax.ShapeDtypeStruct((N, 1), jnp.float32),
            jax.ShapeDtypeStruct((N, 1), jnp.float32),
        ],
    )(x, Wg)


def _ffn_body(gid_ref, nblk_ref, xs_ref, w1_ref, b1_ref, w2_ref, b2_ref,
              w_ref, ys_ref):
    g = pl.program_id(0)

    @pl.when(g < nblk_ref[0])
    def _():
        xb = xs_ref[...].astype(jnp.bfloat16)
        h = jnp.dot(xb, w1_ref[0].astype(jnp.bfloat16),
                    preferred_element_type=jnp.float32)
        h = jnp.maximum(h + b1_ref[0], 0.0).astype(jnp.bfloat16)
        y = jnp.dot(h, w2_ref[0].astype(jnp.bfloat16),
                    preferred_element_type=jnp.float32)
        ys_ref[...] = (y + b2_ref[0]) * w_ref[...]


def _ffn(xs, W1, b1, W2, b2, w_col, gid, nblk):
    grid_spec = pltpu.PrefetchScalarGridSpec(
        num_scalar_prefetch=2,
        grid=(G,),
        in_specs=[
            pl.BlockSpec((BLK, D), lambda g, gid, nblk: (g, 0)),
            pl.BlockSpec((1, D, F), lambda g, gid, nblk: (gid[g], 0, 0)),
            pl.BlockSpec((1, 1, F), lambda g, gid, nblk: (gid[g], 0, 0)),
            pl.BlockSpec((1, F, D), lambda g, gid, nblk: (gid[g], 0, 0)),
            pl.BlockSpec((1, 1, D), lambda g, gid, nblk: (gid[g], 0, 0)),
            pl.BlockSpec((BLK, 1), lambda g, gid, nblk: (g, 0)),
        ],
        out_specs=pl.BlockSpec((BLK, D), lambda g, gid, nblk: (g, 0)),
    )
    return pl.pallas_call(
        _ffn_body,
        grid_spec=grid_spec,
        out_shape=jax.ShapeDtypeStruct((RPAD, D), jnp.float32),
    )(gid, nblk, xs, W1, b1.reshape(E, 1, F), W2, b2.reshape(E, 1, D), w_col)


_NC, _NS = 2, 16
_NW = _NC * _NS

_G_RPW = RPAD // _NW
_G_CH = 40
_G_NCH = _G_RPW // _G_CH

_C_TPW = N // _NW
_C_CH = 32

@functools.cache
def _sc_kernels():
    mesh = plsc.VectorSubcoreMesh(
        core_axis_name="c", subcore_axis_name="s", num_cores=_NC)

    @functools.partial(
        pl.kernel,
        mesh=mesh,
        out_type=jax.ShapeDtypeStruct((RPAD, D), jnp.float32),
        scratch_types=[
            pltpu.VMEM((_G_RPW,), jnp.int32),
            pltpu.VMEM((_G_CH, D), jnp.float32),
            pltpu.VMEM((_G_CH, D), jnp.float32),
            pltpu.SemaphoreType.DMA,
            pltpu.SemaphoreType.DMA,
            pltpu.SemaphoreType.DMA,
            pltpu.SemaphoreType.DMA,
        ],
    )
    def sc_gather(tok_hbm, x_hbm, xs_hbm, idx_v, r0, r1, sg0, sg1, sw0, sw1):
        wid = lax.axis_index("s") * _NC + lax.axis_index("c")
        base = wid * _G_RPW
        pltpu.sync_copy(tok_hbm.at[pl.ds(base, _G_RPW)], idx_v)
        rows, sg, sw = [r0, r1], [sg0, sg1], [sw0, sw1]
        gcp, wcp = [None] * _G_NCH, [None] * _G_NCH
        gcp[0] = pltpu.async_copy(
            x_hbm.at[idx_v.at[pl.ds(0, _G_CH)]], rows[0], sg[0])
        for t in range(_G_NCH):
            if t >= 1:
                wcp[t - 1].wait()
            if t + 1 < _G_NCH:
                gcp[t + 1] = pltpu.async_copy(
                    x_hbm.at[idx_v.at[pl.ds((t + 1) * _G_CH, _G_CH)]],
                    rows[(t + 1) % 2], sg[(t + 1) % 2])
            gcp[t].wait()
            wcp[t] = pltpu.async_copy(
                rows[t % 2], xs_hbm.at[pl.ds(base + t * _G_CH, _G_CH)],
                sw[t % 2])
        wcp[_G_NCH - 1].wait()

    @functools.partial(
        pl.kernel,
        mesh=mesh,
        out_type=jax.ShapeDtypeStruct((N, D), jnp.float32),
        scratch_types=[
            pltpu.VMEM((_C_CH,), jnp.int32),
            pltpu.VMEM((_C_CH,), jnp.int32),
            pltpu.VMEM((_C_CH, D), jnp.float32),
            pltpu.VMEM((_C_CH, D), jnp.float32),
            pltpu.SemaphoreType.DMA,
            pltpu.SemaphoreType.DMA,
        ],
    )
    def sc_combine(p0_hbm, p1_hbm, ys_hbm, out_hbm,
                   i0_v, i1_v, a_v, b_v, s0, s1):
        wid = lax.axis_index("s") * _NC + lax.axis_index("c")
        for t in range(_C_TPW // _C_CH):
            base = wid * _C_TPW + t * _C_CH
            pltpu.sync_copy(p0_hbm.at[pl.ds(base, _C_CH)], i0_v)
            pltpu.sync_copy(p1_hbm.at[pl.ds(base, _C_CH)], i1_v)
            cp0 = pltpu.async_copy(ys_hbm.at[i0_v], a_v, s0)
            cp1 = pltpu.async_copy(ys_hbm.at[i1_v], b_v, s1)
            cp0.wait()
            cp1.wait()

            def _row(r, carry):
                def _col(c, cc):
                    sl = pl.ds(c * 16, 16)
                    a_v[r, sl] = a_v[r, sl] + b_v[r, sl]
                    return cc
                return lax.fori_loop(0, D // 16, _col, carry, unroll=4)

            lax.fori_loop(0, _C_CH, _row, 0)
            pltpu.sync_copy(a_v, out_hbm.at[pl.ds(base, _C_CH)])

    return sc_gather, sc_combine


def _dispatch(topi, wn):
    e_flat = topi.reshape(-1)
    w_flat = wn.reshape(-1)
    onehot = (e_flat[:, None] == jnp.arange(E, dtype=e_flat.dtype))
    oh_i = onehot.astype(jnp.int32)
    counts = jnp.sum(oh_i, axis=0)
    ranks = jnp.sum(jnp.where(onehot, jnp.cumsum(oh_i, axis=0) - 1, 0), axis=1)
    nb = (counts + BLK - 1) // BLK
    cum_nb = jnp.cumsum(nb)
    rowstart = BLK * (cum_nb - nb)
    dst = rowstart[e_flat] + ranks
    tok = jnp.zeros((RPAD,), jnp.int32).at[dst].set(
        jnp.arange(N * K, dtype=jnp.int32) // K)
    w_pad = jnp.zeros((RPAD,), jnp.float32).at[dst].set(w_flat)
    gid = jnp.minimum(
        jnp.searchsorted(cum_nb, jnp.arange(G, dtype=jnp.int32), side="right"),
        E - 1).astype(jnp.int32)
    nblk = cum_nb[E - 1:].astype(jnp.int32)
    p = dst.reshape(N, K)
    return tok, w_pad, gid, nblk, p[:, 0], p[:, 1]


def kernel(x, Wg, W1, b1, W2, b2):
    probs, topv, topi, wn, ent, mass = _router(x, Wg)
    tok, w_pad, gid, nblk, p0, p1 = _dispatch(topi, wn)
    sc_gather, sc_combine = _sc_kernels()
    xs = sc_gather(tok, x)
    ys = _ffn(xs, W1, b1, W2, b2, w_pad.reshape(RPAD, 1), gid, nblk)
    out = sc_combine(p0, p1, ys)
    return (out, probs, topi, topv, ent.reshape(N), mass.reshape(N))


# speedup vs baseline: 1.2824x; 1.0158x over previous
import functools

import jax
import jax.numpy as jnp
from jax import lax
from jax.experimental import pallas as pl
from jax.experimental.pallas import tpu as pltpu
from jax.experimental.pallas import tpu_sc as plsc

N, D, E, K, F = 2048, 1024, 8, 2, 2048
TEMP = 1.0

BT = 256
BLK = 128
G = 40
RPAD = G * BLK


def _router_body(x_ref, wg_ref, probs_ref, topv_ref, topi_ref, wn_ref,
                 ent_ref, mass_ref, ranks_ref, counts_ref, cnt):
    i = pl.program_id(0)

    @pl.when(i == 0)
    def _():
        cnt[...] = jnp.zeros_like(cnt)

    l = jnp.dot(x_ref[...], wg_ref[...], preferred_element_type=jnp.float32)
    l = l / TEMP
    m = jnp.max(l, axis=1, keepdims=True)
    e = jnp.exp(l - m)
    s = jnp.sum(e, axis=1, keepdims=True)
    p = e / s
    probs_ref[...] = p
    iota = lax.broadcasted_iota(jnp.int32, p.shape, 1)
    v1 = jnp.max(p, axis=1, keepdims=True)
    i1 = jnp.min(jnp.where(p == v1, iota, E), axis=1, keepdims=True)
    p2 = jnp.where(iota == i1, -1.0, p)
    v2 = jnp.max(p2, axis=1, keepdims=True)
    i2 = jnp.min(jnp.where(p2 == v2, iota, E), axis=1, keepdims=True)
    topv_ref[...] = jnp.concatenate([v1, v2], axis=1)
    topi_ref[...] = jnp.concatenate([i1, i2], axis=1)
    ws = v1 + v2
    wn_ref[...] = jnp.concatenate([v1 / ws, v2 / ws], axis=1)
    ent_ref[...] = -jnp.sum(p * jnp.log(p + 1e-9), axis=1, keepdims=True)
    mass_ref[...] = ws
    oh = (i1 == iota).astype(jnp.int32) + (i2 == iota).astype(jnp.int32)
    tri = (lax.broadcasted_iota(jnp.int32, (BT, BT), 0)
           > lax.broadcasted_iota(jnp.int32, (BT, BT), 1)).astype(jnp.float32)
    excl = jnp.dot(tri, oh.astype(jnp.float32),
                   preferred_element_type=jnp.float32,
                   precision=lax.Precision.HIGHEST).astype(jnp.int32) + cnt[...]
    r1 = jnp.sum(jnp.where(i1 == iota, excl, 0), axis=1, keepdims=True)
    r2 = jnp.sum(jnp.where(i2 == iota, excl, 0), axis=1, keepdims=True)
    ranks_ref[...] = jnp.concatenate([r1, r2], axis=1)
    cnt[...] = cnt[...] + jnp.sum(oh, axis=0, keepdims=True)
    counts_ref[...] = cnt[...]


def _router(x, Wg):
    return pl.pallas_call(
        _router_body,
        grid=(N // BT,),
        in_specs=[
            pl.BlockSpec((BT, D), lambda i: (i, 0)),
            pl.BlockSpec((D, E), lambda i: (0, 0)),
        ],
        out_specs=[
            pl.BlockSpec((BT, E), lambda i: (i, 0)),
            pl.BlockSpec((BT, K), lambda i: (i, 0)),
            pl.BlockSpec((BT, K), lambda i: (i, 0)),
            pl.BlockSpec((BT, K), lambda i: (i, 0)),
            pl.BlockSpec((BT, 1), lambda i: (i, 0)),
            pl.BlockSpec((BT, 1), lambda i: (i, 0)),
            pl.BlockSpec((BT, K), lambda i: (i, 0)),
            pl.BlockSpec((1, E), lambda i: (0, 0)),
        ],
        out_shape=[
            jax.ShapeDtypeStruct((N, E), jnp.float32),
            jax.ShapeDtypeStruct((N, K), jnp.float32),
            jax.ShapeDtypeStruct((N, K), jnp.int32),
            jax.ShapeDtypeStruct((N, K), jnp.float32),
            jax.ShapeDtypeStruct((N, 1), jnp.float32),
            jax.ShapeDtypeStruct((N, 1), jnp.float32),
            jax.ShapeDtypeStruct((N, K), jnp.int32),
            jax.ShapeDtypeStruct((1, E), jnp.int32),
        ],
        scratch_shapes=[pltpu.VMEM((1, E), jnp.int32)],
    )(x, Wg)


def _ffn_body(gid_ref, nblk_ref, xs_ref, w1_ref, b1_ref, w2_ref, b2_ref,
              w_ref, ys_ref):
    g = pl.program_id(0)

    @pl.when(g < nblk_ref[0])
    def _():
        xb = xs_ref[...].astype(jnp.bfloat16)
        h = jnp.dot(xb, w1_ref[0].astype(jnp.bfloat16),
                    preferred_element_type=jnp.float32)
        h = jnp.maximum(h + b1_ref[0], 0.0).astype(jnp.bfloat16)
        y = jnp.dot(h, w2_ref[0].astype(jnp.bfloat16),
                    preferred_element_type=jnp.float32)
        ys_ref[...] = (y + b2_ref[0]) * w_ref[...]


def _ffn(xs, W1, b1, W2, b2, w_col, gid, nblk):
    grid_spec = pltpu.PrefetchScalarGridSpec(
        num_scalar_prefetch=2,
        grid=(G,),
        in_specs=[
            pl.BlockSpec((BLK, D), lambda g, gid, nblk: (g, 0)),
            pl.BlockSpec((1, D, F), lambda g, gid, nblk: (gid[g], 0, 0)),
            pl.BlockSpec((1, 1, F), lambda g, gid, nblk: (gid[g], 0, 0)),
            pl.BlockSpec((1, F, D), lambda g, gid, nblk: (gid[g], 0, 0)),
            pl.BlockSpec((1, 1, D), lambda g, gid, nblk: (gid[g], 0, 0)),
            pl.BlockSpec((BLK, 1), lambda g, gid, nblk: (g, 0)),
        ],
        out_specs=pl.BlockSpec((BLK, D), lambda g, gid, nblk: (g, 0)),
    )
    return pl.pallas_call(
        _ffn_body,
        grid_spec=grid_spec,
        out_shape=jax.ShapeDtypeStruct((RPAD, D), jnp.float32),
    )(gid, nblk, xs, W1, b1.reshape(E, 1, F), W2, b2.reshape(E, 1, D), w_col)


_NC, _NS = 2, 16
_NW = _NC * _NS

_G_RPW = RPAD // _NW
_G_CH = 40
_G_NCH = _G_RPW // _G_CH

_C_TPW = N // _NW
_C_CH = 32

@functools.cache
def _sc_kernels():
    mesh = plsc.VectorSubcoreMesh(
        core_axis_name="c", subcore_axis_name="s", num_cores=_NC)

    @functools.partial(
        pl.kernel,
        mesh=mesh,
        out_type=jax.ShapeDtypeStruct((RPAD, D), jnp.float32),
        scratch_types=[
            pltpu.VMEM((_G_RPW,), jnp.int32),
            pltpu.VMEM((_G_CH, D), jnp.float32),
            pltpu.VMEM((_G_CH, D), jnp.float32),
            pltpu.SemaphoreType.DMA,
            pltpu.SemaphoreType.DMA,
            pltpu.SemaphoreType.DMA,
            pltpu.SemaphoreType.DMA,
        ],
    )
    def sc_gather(tok_hbm, x_hbm, xs_hbm, idx_v, r0, r1, sg0, sg1, sw0, sw1):
        wid = lax.axis_index("s") * _NC + lax.axis_index("c")
        base = wid * _G_RPW
        pltpu.sync_copy(tok_hbm.at[pl.ds(base, _G_RPW)], idx_v)
        rows, sg, sw = [r0, r1], [sg0, sg1], [sw0, sw1]
        gcp, wcp = [None] * _G_NCH, [None] * _G_NCH
        gcp[0] = pltpu.async_copy(
            x_hbm.at[idx_v.at[pl.ds(0, _G_CH)]], rows[0], sg[0])
        for t in range(_G_NCH):
            if t >= 1:
                wcp[t - 1].wait()
            if t + 1 < _G_NCH:
                gcp[t + 1] = pltpu.async_copy(
                    x_hbm.at[idx_v.at[pl.ds((t + 1) * _G_CH, _G_CH)]],
                    rows[(t + 1) % 2], sg[(t + 1) % 2])
            gcp[t].wait()
            wcp[t] = pltpu.async_copy(
                rows[t % 2], xs_hbm.at[pl.ds(base + t * _G_CH, _G_CH)],
                sw[t % 2])
        wcp[_G_NCH - 1].wait()

    @functools.partial(
        pl.kernel,
        mesh=mesh,
        out_type=jax.ShapeDtypeStruct((N, D), jnp.float32),
        scratch_types=[
            pltpu.VMEM((_C_CH,), jnp.int32),
            pltpu.VMEM((_C_CH,), jnp.int32),
            pltpu.VMEM((_C_CH, D), jnp.float32),
            pltpu.VMEM((_C_CH, D), jnp.float32),
            pltpu.SemaphoreType.DMA,
            pltpu.SemaphoreType.DMA,
        ],
    )
    def sc_combine(p0_hbm, p1_hbm, ys_hbm, out_hbm,
                   i0_v, i1_v, a_v, b_v, s0, s1):
        wid = lax.axis_index("s") * _NC + lax.axis_index("c")
        for t in range(_C_TPW // _C_CH):
            base = wid * _C_TPW + t * _C_CH
            pltpu.sync_copy(p0_hbm.at[pl.ds(base, _C_CH)], i0_v)
            pltpu.sync_copy(p1_hbm.at[pl.ds(base, _C_CH)], i1_v)
            cp0 = pltpu.async_copy(ys_hbm.at[i0_v], a_v, s0)
            cp1 = pltpu.async_copy(ys_hbm.at[i1_v], b_v, s1)
            cp0.wait()
            cp1.wait()

            def _row(r, carry):
                def _col(c, cc):
                    sl = pl.ds(c * 16, 16)
                    a_v[r, sl] = a_v[r, sl] + b_v[r, sl]
                    return cc
                return lax.fori_loop(0, D // 16, _col, carry, unroll=4)

            lax.fori_loop(0, _C_CH, _row, 0)
            pltpu.sync_copy(a_v, out_hbm.at[pl.ds(base, _C_CH)])

    return sc_gather, sc_combine


def _dispatch(topi, wn, ranks, counts):
    e_flat = topi.reshape(-1)
    nb = (counts[0] + BLK - 1) // BLK
    cum_nb = jnp.cumsum(nb)
    rowstart = BLK * (cum_nb - nb)
    dst = rowstart[e_flat] + ranks.reshape(-1)
    payload = jnp.stack(
        [jnp.arange(N * K, dtype=jnp.int32) // K,
         lax.bitcast_convert_type(wn.reshape(-1), jnp.int32)], axis=0)
    z = jnp.zeros((2, RPAD), jnp.int32).at[:, dst].set(payload)
    tok = z[0]
    w_pad = lax.bitcast_convert_type(z[1], jnp.float32)
    gid = jnp.minimum(
        jnp.searchsorted(cum_nb, jnp.arange(G, dtype=jnp.int32), side="right"),
        E - 1).astype(jnp.int32)
    nblk = cum_nb[E - 1:].astype(jnp.int32)
    p = dst.reshape(N, K)
    return tok, w_pad, gid, nblk, p[:, 0], p[:, 1]


def kernel(x, Wg, W1, b1, W2, b2):
    probs, topv, topi, wn, ent, mass, ranks, counts = _router(x, Wg)
    tok, w_pad, gid, nblk, p0, p1 = _dispatch(topi, wn, ranks, counts)
    sc_gather, sc_combine = _sc_kernels()
    xs = sc_gather(tok, x)
    ys = _ffn(xs, W1, b1, W2, b2, w_pad.reshape(RPAD, 1), gid, nblk)
    out = sc_combine(p0, p1, ys)
    return (out, probs, topi, topv, ent.reshape(N), mass.reshape(N))
